# baseline (speedup 1.0000x reference)
import functools
import math

import jax
import jax.numpy as jnp
import numpy as np
from jax import lax
from jax.experimental import pallas as pl
from jax.experimental.pallas import tpu as pltpu
from jax.experimental.pallas import tpu_sc as plsc

B = 4
L = 4096
D = 1024
N_ROWS = B * L
SCALE = math.sqrt(64.0)

_pos = np.arange(L, dtype=np.float32)[:, None]
_div = np.exp(
    np.arange(0, D, 2, dtype=np.float32) * (-math.log(10000.0) / D)
).astype(np.float32)
_PE = np.zeros((L, D), dtype=np.float32)
_PE[:, 0::2] = np.sin(_pos * _div)
_PE[:, 1::2] = np.cos(_pos * _div)

NC, NS, LANES = 2, 16, 16
NW = NC * NS
L_PER_W = L // NW
P = 4
RPS = B * P
N_STEPS = L_PER_W // P
VREGS_PER_ROW = D // LANES
NBUF = 4


@functools.cache
def _build():
    @functools.partial(
        pl.kernel,
        mesh=plsc.VectorSubcoreMesh(core_axis_name="c", subcore_axis_name="s"),
        out_type=jax.ShapeDtypeStruct((N_ROWS, D), jnp.float32),
        scratch_types=[
            pltpu.VMEM((N_STEPS * RPS,), jnp.int32),
            pltpu.VMEM((RPS, D), jnp.float32),
            pltpu.VMEM((RPS, D), jnp.float32),
            pltpu.VMEM((RPS, D), jnp.float32),
            pltpu.VMEM((RPS, D), jnp.float32),
            pltpu.VMEM((P, D), jnp.float32),
            pltpu.VMEM((P, D), jnp.float32),
            pltpu.SemaphoreType.DMA,
            pltpu.SemaphoreType.DMA,
            pltpu.SemaphoreType.DMA,
            pltpu.SemaphoreType.DMA,
            pltpu.SemaphoreType.DMA,
            pltpu.SemaphoreType.DMA,
            pltpu.SemaphoreType.DMA,
            pltpu.SemaphoreType.DMA,
            pltpu.SemaphoreType.DMA,
            pltpu.SemaphoreType.DMA,
        ],
    )
    def _emb_pe_kernel(
        src_hbm, table_hbm, pe_hbm, out_hbm,
        idx_all, r0, r1, r2, r3, pe0, pe1,
        g0, g1, g2, g3, s0, s1, s2, s3, p0, p1,
    ):
        wid = lax.axis_index("s") * NC + lax.axis_index("c")
        base_l = wid * L_PER_W
        row = (r0, r1, r2, r3)
        peb = (pe0, pe1)
        gs = (g0, g1, g2, g3)
        ss = (s0, s1, s2, s3)
        ps = (p0, p1)

        def idx_slice(c):
            return idx_all.at[pl.ds(c * RPS, RPS)]

        def issue_gather(c, q):
            pltpu.async_copy(table_hbm.at[idx_slice(c)], row[q], gs[q])

        def wait_gather(c, q):
            pltpu.make_async_copy(
                table_hbm.at[idx_slice(c)], row[q], gs[q]
            ).wait()

        def issue_pe(c, q):
            pltpu.async_copy(
                pe_hbm.at[pl.ds(base_l + c * P, P)], peb[q], ps[q]
            )

        def wait_pe(c, q):
            pltpu.make_async_copy(
                pe_hbm.at[pl.ds(base_l + c * P, P)], peb[q], ps[q]
            ).wait()

        def issue_stores(c, q):
            for b in range(B):
                pltpu.async_copy(
                    row[q].at[pl.ds(b * P, P)],
                    out_hbm.at[pl.ds(b * L + base_l + c * P, P)],
                    ss[q],
                )

        def wait_stores(q):
            for _ in range(B):
                pltpu.make_async_copy(
                    row[q].at[pl.ds(0, P)], out_hbm.at[pl.ds(0, P)], ss[q]
                ).wait()

        def emit_step(c, q, qpe, first):
            qn = (q + 2) % NBUF
            if not (first and q < 2):
                wait_stores(qn)
            issue_gather((c + 2) % N_STEPS, qn)
            issue_pe((c + 1) % N_STEPS, 1 - qpe)
            wait_pe(c, qpe)
            wait_gather(c, q)

            def _col(j, _):
                sl = pl.ds(j * LANES, LANES)
                for p in range(P):
                    pe_v = peb[qpe][p, sl]
                    for b in range(B):
                        r = b * P + p
                        row[q][r, sl] = row[q][r, sl] * SCALE + pe_v
                return _

            lax.fori_loop(0, VREGS_PER_ROW, _col, 0, unroll=4)
            issue_stores(c, q)

        def emit_group(c4, first):
            for u in range(NBUF):
                emit_step(c4 * NBUF + u, u, u % 2, first)

        pltpu.sync_copy(
            src_hbm.at[pl.ds(wid * (N_STEPS * RPS), N_STEPS * RPS)], idx_all
        )
        issue_gather(0, 0)
        issue_gather(1, 1)
        issue_pe(0, 0)

        emit_group(0, True)

        def _c4_body(c4, _):
            emit_group(c4, False)
            return _

        lax.fori_loop(1, N_STEPS // NBUF, _c4_body, 0)

        wait_gather(0, 0)
        wait_gather(1, 1)
        wait_pe(0, 0)
        wait_stores(2)
        wait_stores(3)

    return _emb_pe_kernel


def kernel(src, emb_table):
    src_arr = (
        src.reshape(B, NW, N_STEPS, P)
        .transpose(1, 2, 0, 3)
        .reshape(N_ROWS)
        .astype(jnp.int32)
    )
    pe = jnp.asarray(_PE)
    out = _build()(src_arr, emb_table, pe)
    return out.reshape(B, L, D)


# speedup vs baseline: 1.9880x; 1.9880x over previous
import functools
import math

import jax
import jax.numpy as jnp
import numpy as np
from jax import lax
from jax.experimental import pallas as pl
from jax.experimental.pallas import tpu as pltpu
from jax.experimental.pallas import tpu_sc as plsc

B = 4
L = 4096
D = 1024
N_ROWS = B * L
SCALE = math.sqrt(64.0)

_pos = np.arange(L, dtype=np.float32)[:, None]
_div = np.exp(
    np.arange(0, D, 2, dtype=np.float32) * (-math.log(10000.0) / D)
).astype(np.float32)
_PE = np.zeros((L, D), dtype=np.float32)
_PE[:, 0::2] = np.sin(_pos * _div)
_PE[:, 1::2] = np.cos(_pos * _div)

NC, NS, LANES = 2, 16, 16
NW = NC * NS
L_PER_W = L // NW
P = 4
RPS = B * P
N_STEPS = L_PER_W // P
VREGS_PER_ROW = D // LANES
NBUF = 4


@functools.cache
def _build():
    @functools.partial(
        pl.kernel,
        mesh=plsc.VectorSubcoreMesh(core_axis_name="c", subcore_axis_name="s"),
        out_type=jax.ShapeDtypeStruct((N_ROWS, D), jnp.float32),
        scratch_types=[
            pltpu.VMEM((N_STEPS * RPS,), jnp.int32),
            pltpu.VMEM((RPS, D), jnp.float32),
            pltpu.VMEM((RPS, D), jnp.float32),
            pltpu.VMEM((RPS, D), jnp.float32),
            pltpu.VMEM((RPS, D), jnp.float32),
            pltpu.VMEM((P, D), jnp.float32),
            pltpu.VMEM((P, D), jnp.float32),
            pltpu.SemaphoreType.DMA,
            pltpu.SemaphoreType.DMA,
            pltpu.SemaphoreType.DMA,
            pltpu.SemaphoreType.DMA,
            pltpu.SemaphoreType.DMA,
            pltpu.SemaphoreType.DMA,
            pltpu.SemaphoreType.DMA,
            pltpu.SemaphoreType.DMA,
            pltpu.SemaphoreType.DMA,
            pltpu.SemaphoreType.DMA,
        ],
    )
    def _emb_pe_kernel(
        src_hbm, table_hbm, pe_hbm, out_hbm,
        idx_all, r0, r1, r2, r3, pe0, pe1,
        g0, g1, g2, g3, s0, s1, s2, s3, p0, p1,
    ):
        wid = lax.axis_index("s") * NC + lax.axis_index("c")
        base_l = wid * L_PER_W
        row = (r0, r1, r2, r3)
        peb = (pe0, pe1)
        gs = (g0, g1, g2, g3)
        ss = (s0, s1, s2, s3)
        ps = (p0, p1)

        def idx_slice(c):
            return idx_all.at[pl.ds(c * RPS, RPS)]

        def issue_gather(c, q):
            pltpu.async_copy(table_hbm.at[idx_slice(c)], row[q], gs[q])

        def wait_gather(c, q):
            pltpu.make_async_copy(
                table_hbm.at[idx_slice(c)], row[q], gs[q]
            ).wait()

        def issue_pe(c, q):
            pltpu.async_copy(
                pe_hbm.at[pl.ds(base_l + c * P, P)], peb[q], ps[q]
            )

        def wait_pe(c, q):
            pltpu.make_async_copy(
                pe_hbm.at[pl.ds(base_l + c * P, P)], peb[q], ps[q]
            ).wait()

        def issue_stores(c, q):
            for b in range(B):
                pltpu.async_copy(
                    row[q].at[pl.ds(b * P, P)],
                    out_hbm.at[pl.ds(b * L + base_l + c * P, P)],
                    ss[q],
                )

        def wait_stores(q):
            for _ in range(B):
                pltpu.make_async_copy(
                    row[q].at[pl.ds(0, P)], out_hbm.at[pl.ds(0, P)], ss[q]
                ).wait()

        def emit_step(c, q, qpe, first):
            qn = (q + 2) % NBUF
            issue_pe((c + 1) % N_STEPS, 1 - qpe)
            if not (first and q < 2):
                wait_stores(qn)
            issue_gather((c + 2) % N_STEPS, qn)
            wait_pe(c, qpe)
            wait_gather(c, q)

            def _col(j, _):
                sl = pl.ds(j * LANES, LANES)
                for p in range(P):
                    pe_v = peb[qpe][p, sl]
                    for b in range(B):
                        r = b * P + p
                        row[q][r, sl] = row[q][r, sl] * SCALE + pe_v
                return _

            lax.fori_loop(0, VREGS_PER_ROW, _col, 0)
            issue_stores(c, q)

        def emit_group(c4, first):
            for u in range(NBUF):
                emit_step(c4 * NBUF + u, u, u % 2, first)

        pltpu.sync_copy(
            src_hbm.at[pl.ds(wid * (N_STEPS * RPS), N_STEPS * RPS)], idx_all
        )
        issue_gather(0, 0)
        issue_gather(1, 1)
        issue_pe(0, 0)

        emit_group(0, True)

        def _c4_body(c4, _):
            emit_group(c4, False)
            return _

        lax.fori_loop(1, N_STEPS // NBUF, _c4_body, 0)

        wait_gather(0, 0)
        wait_gather(1, 1)
        wait_pe(0, 0)
        wait_stores(2)
        wait_stores(3)

    return _emb_pe_kernel


def kernel(src, emb_table):
    src_arr = (
        src.reshape(B, NW, N_STEPS, P)
        .transpose(1, 2, 0, 3)
        .reshape(N_ROWS)
        .astype(jnp.int32)
    )
    pe = jnp.asarray(_PE)
    out = _build()(src_arr, emb_table, pe)
    return out.reshape(B, L, D)
